# Initial kernel scaffold; baseline (speedup 1.0000x reference)
import jax
import jax.numpy as jnp
from jax.experimental import pallas as pl


def kernel(img):
    raise NotImplementedError("write your pallas kernel here")


# speedup vs baseline: 1.4180x; 1.4180x over previous
import functools

import jax
import jax.numpy as jnp
from jax import lax
from jax.experimental import pallas as pl
from jax.experimental.pallas import tpu as pltpu
from jax.experimental.pallas import tpu_sc as plsc

C, H, W = 192, 512, 512
R = C * H
NC, NS = 2, 16
NW = NC * NS
RW = R // NW
CHUNK = 96
NCHUNK = RW // CHUNK

_mesh = plsc.VectorSubcoreMesh(core_axis_name="c", subcore_axis_name="s")


@functools.partial(
    pl.kernel,
    out_type=jax.ShapeDtypeStruct((R, W), jnp.float32),
    mesh=_mesh,
    scratch_types=[
        pltpu.VMEM((NCHUNK, CHUNK), jnp.int32),
        pltpu.VMEM((CHUNK, W), jnp.float32),
        pltpu.SemaphoreType.DMA,
    ],
)
def _permute_rows(img_hbm, idx_hbm, out_hbm, idx_v, buf, sem):
    wid = lax.axis_index("s") * NC + lax.axis_index("c")
    pltpu.sync_copy(idx_hbm.at[wid], idx_v)
    base = wid * RW

    def body(k, carry):
        pltpu.async_copy(img_hbm.at[idx_v.at[k]], buf, sem).wait()
        pltpu.sync_copy(buf, out_hbm.at[pl.ds(base + k * CHUNK, CHUNK), :])
        return carry

    lax.fori_loop(0, NCHUNK, body, 0)


def kernel(img):
    perm = jax.random.permutation(jax.random.key(42), C)
    row_idx = (perm[:, None] * H + jnp.arange(H)[None, :]).astype(jnp.int32)
    idx = row_idx.reshape(NW, NCHUNK, CHUNK)
    out2 = _permute_rows(img.reshape(R, W), idx)
    return out2.reshape(C, H, W)


# speedup vs baseline: 1.5612x; 1.1010x over previous
import functools

import jax
import jax.numpy as jnp
from jax import lax
from jax.experimental import pallas as pl
from jax.experimental.pallas import tpu as pltpu
from jax.experimental.pallas import tpu_sc as plsc

C, H, W = 192, 512, 512
R = C * H
NC, NS = 2, 16
NW = NC * NS
RW = R // NW
CHUNK = 64
NCHUNK = RW // CHUNK
NBUF = 3
NGRP = NCHUNK // NBUF

_mesh = plsc.VectorSubcoreMesh(core_axis_name="c", subcore_axis_name="s")


@functools.partial(
    pl.kernel,
    out_type=jax.ShapeDtypeStruct((R, W), jnp.float32),
    mesh=_mesh,
    scratch_types=[
        pltpu.VMEM((NCHUNK, CHUNK), jnp.int32),
        [pltpu.VMEM((CHUNK, W), jnp.float32)] * NBUF,
        [pltpu.SemaphoreType.DMA] * NBUF,
        [pltpu.SemaphoreType.DMA] * NBUF,
    ],
)
def _permute_rows(img_hbm, idx_hbm, out_hbm, idx_v, bufs, isems, osems):
    wid = lax.axis_index("s") * NC + lax.axis_index("c")
    pltpu.sync_copy(idx_hbm.at[wid], idx_v)
    base = wid * RW

    def start_in(k, b):
        pltpu.make_async_copy(img_hbm.at[idx_v.at[k]], bufs[b], isems[b]).start()

    def wait_in(b):
        pltpu.make_async_copy(img_hbm.at[idx_v.at[0]], bufs[b], isems[b]).wait()

    def start_out(k, b):
        dst = out_hbm.at[pl.ds(base + k * CHUNK, CHUNK), :]
        pltpu.make_async_copy(bufs[b], dst, osems[b]).start()

    def wait_out(b):
        dst = out_hbm.at[pl.ds(base, CHUNK), :]
        pltpu.make_async_copy(bufs[b], dst, osems[b]).wait()

    start_in(0, 0)
    start_in(1, 1)

    def body(g, carry):
        for b in range(NBUF):
            k = g * NBUF + b
            b2 = (b + 2) % NBUF
            wait_in(b)
            start_out(k, b)
            pl.when(k >= 1)(lambda: wait_out(b2))
            pl.when(k + 2 < NCHUNK)(lambda: start_in(k + 2, b2))
        return carry

    lax.fori_loop(0, NGRP, body, 0)
    wait_out((NCHUNK - 1) % NBUF)


def kernel(img):
    perm = jax.random.permutation(jax.random.key(42), C)
    row_idx = (perm[:, None] * H + jnp.arange(H)[None, :]).astype(jnp.int32)
    idx = row_idx.reshape(NW, NCHUNK, CHUNK)
    out2 = _permute_rows(img.reshape(R, W), idx)
    return out2.reshape(C, H, W)
